# baseline (speedup 1.0000x reference)
import jax
import jax.numpy as jnp
from jax import lax
from jax.experimental import pallas as pl
from jax.experimental.pallas import tpu as pltpu
from jax.experimental.pallas import tpu_sc as plsc

D = 64
SCALE = 8.0
NC = 2
NS = 16
NW = NC * NS
B_TOT = 4096 * 200
B_PER_W = B_TOT // NW
CHUNK = 400
NBUF = 2
NCHUNK = B_PER_W // CHUNK
SPLITS = [(0, 128), (128, 128), (256, 128), (384, 16)]
LANES = 16


def _emb_body(table_hbm, idx_hbm, out_hbm,
              idx_all, rows0, rows1, obuf0, obuf1,
              gsem0, gsem1, ssem0, ssem1):
    rows_v = (rows0, rows1)
    obuf_v = (obuf0, obuf1)
    gsems = (gsem0, gsem1)
    ssems = (ssem0, ssem1)

    wid = lax.axis_index("s") * NC + lax.axis_index("c")
    base = wid * B_PER_W

    def gather_descs(g, b):
        return [
            pltpu.make_async_copy(
                table_hbm.at[idx_all.at[pl.ds(g * CHUNK + off, ln)]],
                rows_v[b].at[pl.ds(off, ln), :],
                gsems[b],
            )
            for off, ln in SPLITS
        ]

    def fire_gather(g, b):
        for d_ in gather_descs(g, b):
            d_.start()

    def drain_gather(g, b):
        for d_ in gather_descs(g, b):
            d_.wait()

    def store_desc(g, b):
        return pltpu.make_async_copy(
            obuf_v[b], out_hbm.at[pl.ds(base + g * CHUNK, CHUNK), :],
            ssems[b])

    def scale(b):
        src = rows_v[b]
        dst = obuf_v[b]

        @pl.loop(0, CHUNK, unroll=8)
        def _(i):
            for j in range(D // LANES):
                sl = (i, pl.ds(j * LANES, LANES))
                dst[sl] = src[sl] * SCALE

    pltpu.sync_copy(idx_hbm.at[pl.ds(base, B_PER_W)], idx_all)

    for b in range(NBUF):
        fire_gather(b, b)
    for b in range(NBUF):
        drain_gather(b, b)
        scale(b)
        store_desc(b, b).start()
        fire_gather(b + NBUF, b)

    @pl.loop(NBUF, NCHUNK - NBUF, step=NBUF)
    def _(g0):
        for b in range(NBUF):
            g = g0 + b
            drain_gather(g, b)
            store_desc(g - NBUF, b).wait()
            scale(b)
            store_desc(g, b).start()
            fire_gather(g + NBUF, b)

    for b in range(NBUF):
        g = NCHUNK - NBUF + b
        drain_gather(g, b)
        store_desc(g - NBUF, b).wait()
        scale(b)
        store_desc(g, b).start()
    for b in range(NBUF):
        store_desc(NCHUNK - NBUF + b, b).wait()


@jax.jit
def _emb_lookup(table, idx):
    mesh = plsc.VectorSubcoreMesh(core_axis_name="c", subcore_axis_name="s")
    f = pl.kernel(
        _emb_body,
        out_type=jax.ShapeDtypeStruct((B_TOT, D), jnp.float32),
        mesh=mesh,
        scratch_types=[
            pltpu.VMEM((B_PER_W,), jnp.int32),
            pltpu.VMEM((CHUNK, D), jnp.float32),
            pltpu.VMEM((CHUNK, D), jnp.float32),
            pltpu.VMEM((CHUNK, D), jnp.float32),
            pltpu.VMEM((CHUNK, D), jnp.float32),
            pltpu.SemaphoreType.DMA,
            pltpu.SemaphoreType.DMA,
            pltpu.SemaphoreType.DMA,
            pltpu.SemaphoreType.DMA,
        ],
        compiler_params=pltpu.CompilerParams(use_tc_tiling_on_sc=False),
    )
    return f(table, idx)


def kernel(x, table):
    idx = x.reshape(-1)
    out = _emb_lookup(table, idx)
    return out.reshape(x.shape + (D,))
